# baseline (speedup 1.0000x reference)
import functools

import jax
import jax.numpy as jnp
from jax import lax
from jax.experimental import pallas as pl
from jax.experimental.pallas import tpu as pltpu
from jax.experimental.pallas import tpu_sc as plsc

N_ROWS = 32000
NUM_CLASSES = 81
PAD_ROWS = 32768
NC, NS = 2, 16
NW = NC * NS
ROWS_PER_W = PAD_ROWS // NW
ELEMS_PER_W = ROWS_PER_W * 4
TBL_ELEMS = N_ROWS * NUM_CLASSES * 4


def _sc_loss_body(cid_hbm, boxes_hbm, ypred_hbm, out_hbm,
                  cid_v, boxes_v, idx_v, preds_v, stage_v, sem):
    c = lax.axis_index("c")
    s = lax.axis_index("s")
    wid = c * NS + s
    base_row = wid * ROWS_PER_W

    pltpu.sync_copy(cid_hbm.at[pl.ds(base_row, ROWS_PER_W)], cid_v)
    for comp in range(4):
        pltpu.sync_copy(
            boxes_hbm.at[pl.ds(comp * PAD_ROWS + base_row, ROWS_PER_W)],
            boxes_v.at[pl.ds(comp * ROWS_PER_W, ROWS_PER_W)],
        )

    iota = lax.iota(jnp.int32, 16)

    def idx_body(v, _):
        cid16 = cid_v[pl.ds(v * 16, 16)]
        grow = base_row + v * 16 + iota
        ridx4 = jnp.minimum((grow * NUM_CLASSES + cid16) * 4, TBL_ELEMS - 4)
        for comp in range(4):
            idx_v[pl.ds(comp * ROWS_PER_W + v * 16, 16)] = ridx4 + comp
        return 0

    lax.fori_loop(0, ROWS_PER_W // 16, idx_body, 0)

    for j in range(ELEMS_PER_W // 128):
        pltpu.async_copy(
            ypred_hbm.at[idx_v.at[pl.ds(j * 128, 128)]],
            preds_v.at[pl.ds(j * 128, 128)],
            sem,
        ).wait()

    def loss_body(v, carry):
        acc, cacc = carry
        cid16 = cid_v[pl.ds(v * 16, 16)]
        m = jnp.minimum(cid16, 1).astype(jnp.float32)
        cacc = cacc + m
        for comp in range(4):
            pred = preds_v[pl.ds(comp * ROWS_PER_W + v * 16, 16)]
            box = boxes_v[pl.ds(comp * ROWS_PER_W + v * 16, 16)]
            diff = jnp.abs(box - pred)
            dlo = jnp.minimum(diff, 1.0)
            loss = 0.5 * dlo * dlo + jnp.maximum(diff, 1.0) - 1.0
            acc = acc + loss * m
        return acc, cacc

    zeros = jnp.zeros((16,), jnp.float32)
    acc, cacc = lax.fori_loop(0, ROWS_PER_W // 16, loss_body, (zeros, zeros))

    stage_v[0, :] = acc
    stage_v[1, :] = cacc * 4.0
    pltpu.sync_copy(stage_v, out_hbm.at[wid])


_sc_loss = functools.partial(
    pl.kernel,
    mesh=plsc.VectorSubcoreMesh(core_axis_name="c", subcore_axis_name="s"),
    out_type=jax.ShapeDtypeStruct((NW, 2, 16), jnp.float32),
    scratch_types=[
        pltpu.VMEM((ROWS_PER_W,), jnp.int32),
        pltpu.VMEM((ELEMS_PER_W,), jnp.float32),
        pltpu.VMEM((ELEMS_PER_W,), jnp.int32),
        pltpu.VMEM((ELEMS_PER_W,), jnp.float32),
        pltpu.VMEM((2, 16), jnp.float32),
        pltpu.SemaphoreType.DMA,
    ],
)(_sc_loss_body)


@jax.jit
def kernel(target_bounding_box, target_class_ids, y_pred):
    cid = target_class_ids.reshape(-1).astype(jnp.int32)
    cid = jnp.pad(cid, (0, PAD_ROWS - N_ROWS))
    boxes = target_bounding_box.reshape(-1, 4)
    boxes = jnp.pad(boxes, ((0, PAD_ROWS - N_ROWS), (0, 0)))
    boxes_soa = boxes.T.reshape(-1)
    ypred_flat = y_pred.reshape(-1)

    out = _sc_loss(cid, boxes_soa, ypred_flat)
    total = jnp.sum(out[:, 0, :])
    cnt = jnp.sum(out[:, 1, :])
    return jnp.where(cnt > 0, total / cnt, jnp.float32(0.0))


# speedup vs baseline: 60.4610x; 60.4610x over previous
import functools

import jax
import jax.numpy as jnp
from jax import lax
from jax.experimental import pallas as pl

BATCH = 16
NUM_ROIS = 2000
NUM_CLASSES = 81
CB = 27
N_CI = NUM_CLASSES // CB


def _loss_body(yt_ref, cid_ref, box_ref, o_ref):
    b = pl.program_id(0)
    ci = pl.program_id(1)

    @pl.when(jnp.logical_and(b == 0, ci == 0))
    def _():
        o_ref[...] = jnp.zeros_like(o_ref)

    blk = yt_ref[0]
    cid = cid_ref[0]
    box = box_ref[0]

    cls_ids = ci * CB + lax.broadcasted_iota(jnp.int32, (CB, 1, NUM_ROIS), 0)
    m = jnp.logical_and(cid[None] == cls_ids, cls_ids > 0)
    mf = m.astype(jnp.float32)

    diff = jnp.abs(box[None] - blk)
    lt1 = (diff < 1.0).astype(jnp.float32)
    loss = lt1 * 0.5 * diff * diff + (1.0 - lt1) * (diff - 0.5)
    s = jnp.sum(loss * mf)

    cnt = jnp.where(
        ci == 0,
        jnp.sum((cid > 0).astype(jnp.float32)) * 4.0,
        0.0,
    )

    lane = lax.broadcasted_iota(jnp.int32, (1, 128), 1)
    upd = jnp.where(lane == 0, s, jnp.where(lane == 1, cnt, 0.0))
    o_ref[...] += upd


@jax.jit
def kernel(target_bounding_box, target_class_ids, y_pred):
    yt = jnp.transpose(y_pred, (0, 2, 3, 1))
    bt = jnp.transpose(target_bounding_box, (0, 2, 1))
    cid3 = target_class_ids.astype(jnp.int32).reshape(BATCH, 1, NUM_ROIS)

    out = pl.pallas_call(
        _loss_body,
        grid=(BATCH, N_CI),
        in_specs=[
            pl.BlockSpec((1, CB, 4, NUM_ROIS), lambda b, ci: (b, ci, 0, 0)),
            pl.BlockSpec((1, 1, NUM_ROIS), lambda b, ci: (b, 0, 0)),
            pl.BlockSpec((1, 4, NUM_ROIS), lambda b, ci: (b, 0, 0)),
        ],
        out_specs=pl.BlockSpec((1, 128), lambda b, ci: (0, 0)),
        out_shape=jax.ShapeDtypeStruct((1, 128), jnp.float32),
    )(yt, cid3, bt)

    total = out[0, 0]
    cnt = out[0, 1]
    return jnp.where(cnt > 0, total / cnt, jnp.float32(0.0))


# speedup vs baseline: 88.6118x; 1.4656x over previous
import functools

import jax
import jax.numpy as jnp
from jax import lax
from jax.experimental import pallas as pl

BATCH = 16
NUM_ROIS = 2000
NUM_CLASSES = 81
CB = 81
N_CI = NUM_CLASSES // CB


def _loss_body(yt_ref, cid_ref, box_ref, o_ref):
    b = pl.program_id(0)

    @pl.when(b == 0)
    def _():
        o_ref[...] = jnp.zeros_like(o_ref)

    blk = yt_ref[0]
    cid = cid_ref[0]
    box = box_ref[0]

    cls_ids = lax.broadcasted_iota(jnp.int32, (CB, 1, NUM_ROIS), 0)
    m = jnp.logical_and(cid[None] == cls_ids, cls_ids > 0)
    mf = m.astype(jnp.float32)

    diff = jnp.abs(box[None] - blk) * mf
    dlo = jnp.minimum(diff, 1.0)
    loss = 0.5 * dlo * dlo + jnp.maximum(diff, 1.0) - 1.0
    s = jnp.sum(loss)

    cnt = jnp.sum((cid > 0).astype(jnp.float32)) * 4.0

    lane = lax.broadcasted_iota(jnp.int32, (1, 128), 1)
    upd = jnp.where(lane == 0, s, jnp.where(lane == 1, cnt, 0.0))
    o_ref[...] += upd


@jax.jit
def kernel(target_bounding_box, target_class_ids, y_pred):
    yt = jnp.transpose(y_pred, (0, 2, 3, 1))
    bt = jnp.transpose(target_bounding_box, (0, 2, 1))
    cid3 = target_class_ids.astype(jnp.int32).reshape(BATCH, 1, NUM_ROIS)

    out = pl.pallas_call(
        _loss_body,
        grid=(BATCH,),
        in_specs=[
            pl.BlockSpec((1, CB, 4, NUM_ROIS), lambda b: (b, 0, 0, 0)),
            pl.BlockSpec((1, 1, NUM_ROIS), lambda b: (b, 0, 0)),
            pl.BlockSpec((1, 4, NUM_ROIS), lambda b: (b, 0, 0)),
        ],
        out_specs=pl.BlockSpec((1, 128), lambda b: (0, 0)),
        out_shape=jax.ShapeDtypeStruct((1, 128), jnp.float32),
    )(yt, cid3, bt)

    total = out[0, 0]
    cnt = out[0, 1]
    return jnp.where(cnt > 0, total / cnt, jnp.float32(0.0))


# speedup vs baseline: 125.0711x; 1.4115x over previous
import functools

import jax
import jax.numpy as jnp
from jax import lax
from jax.experimental import pallas as pl

BATCH = 16
NUM_ROIS = 2000
NUM_CLASSES = 81


def _loss_body(yt_ref, cid_ref, box_ref, o_ref):
    b = pl.program_id(0)

    @pl.when(b == 0)
    def _():
        o_ref[...] = jnp.zeros_like(o_ref)

    blk = yt_ref[0]
    cid4 = cid_ref[0]
    box = box_ref[0]

    cls_ids = lax.broadcasted_iota(jnp.int32, (NUM_CLASSES, 1, 1), 0)
    m = cid4[None] == cls_ids
    pred = jnp.sum(jnp.where(m, blk, 0.0), axis=0)

    mrow = jnp.minimum(cid4, 1).astype(jnp.float32)
    diff = jnp.abs(box - pred) * mrow
    dlo = jnp.minimum(diff, 1.0)
    loss = 0.5 * dlo * dlo + jnp.maximum(diff, 1.0) - 1.0
    s = jnp.sum(loss)
    cnt = jnp.sum(mrow)

    lane = lax.broadcasted_iota(jnp.int32, (1, 128), 1)
    upd = jnp.where(lane == 0, s, jnp.where(lane == 1, cnt, 0.0))
    o_ref[...] += upd


@jax.jit
def kernel(target_bounding_box, target_class_ids, y_pred):
    yt = jnp.transpose(y_pred, (0, 2, 3, 1))
    bt = jnp.transpose(target_bounding_box, (0, 2, 1))
    cid = target_class_ids.astype(jnp.int32)
    cid4 = jnp.broadcast_to(cid[:, None, :], (BATCH, 4, NUM_ROIS))

    out = pl.pallas_call(
        _loss_body,
        grid=(BATCH,),
        in_specs=[
            pl.BlockSpec((1, NUM_CLASSES, 4, NUM_ROIS),
                         lambda b: (b, 0, 0, 0)),
            pl.BlockSpec((1, 4, NUM_ROIS), lambda b: (b, 0, 0)),
            pl.BlockSpec((1, 4, NUM_ROIS), lambda b: (b, 0, 0)),
        ],
        out_specs=pl.BlockSpec((1, 128), lambda b: (0, 0)),
        out_shape=jax.ShapeDtypeStruct((1, 128), jnp.float32),
    )(yt, cid4, bt)

    total = out[0, 0]
    cnt = out[0, 1]
    return jnp.where(cnt > 0, total / cnt, jnp.float32(0.0))
